# Initial kernel scaffold; baseline (speedup 1.0000x reference)
import jax
import jax.numpy as jnp
from jax.experimental import pallas as pl


def kernel(x, edge_index, Wl0, bl0, Wr0, br0, Wl1, bl1, Wr1, br1):
    raise NotImplementedError("write your pallas kernel here")


# speedup vs baseline: 5.0180x; 5.0180x over previous
import functools

import jax
import jax.numpy as jnp
from jax import lax
from jax.experimental import pallas as pl
from jax.experimental.pallas import tpu as pltpu
from jax.experimental.pallas import tpu_sc as plsc

_NC = 2
_NS = 16
_C = 128
_RZ = 64


@functools.lru_cache(maxsize=None)
def _make_agg(n_pad, f, j_chunks, with_count):
  mesh = plsc.VectorSubcoreMesh(core_axis_name="c", subcore_axis_name="s")
  rp = n_pad // _NS

  out_type = [jax.ShapeDtypeStruct((_NC, n_pad, f), jnp.float32)]
  scratch = [
      pltpu.VMEM((j_chunks, _C), jnp.int32),
      pltpu.VMEM((j_chunks, _C), jnp.int32),
      pltpu.VMEM((_C, f), jnp.float32),
      pltpu.VMEM((_RZ, f), jnp.float32),
      pltpu.VMEM_SHARED((n_pad, f), jnp.float32),
      pltpu.SemaphoreType.DMA,
  ]
  if with_count:
    out_type.append(jax.ShapeDtypeStruct((_NC, n_pad), jnp.float32))
    scratch += [
        pltpu.VMEM((rp,), jnp.float32),
        pltpu.VMEM((_C,), jnp.float32),
        pltpu.VMEM_SHARED((n_pad,), jnp.float32),
    ]

  def body(*refs):
    if with_count:
      (x_hbm, srcs_hbm, dsts_hbm, za_hbm, zc_hbm, oc_hbm,
       sums_hbm, cnts_hbm,
       src_v, dst_v, rows_v, zbuf_v, acc_sh, sem,
       zcnt_v, ones_v, cnt_sh) = refs
    else:
      (x_hbm, srcs_hbm, dsts_hbm, za_hbm,
       sums_hbm,
       src_v, dst_v, rows_v, zbuf_v, acc_sh, sem) = refs
    c = lax.axis_index("c")
    s = lax.axis_index("s")
    w = c * _NS + s

    pltpu.sync_copy(srcs_hbm.at[w], src_v)
    pltpu.sync_copy(dsts_hbm.at[w], dst_v)

    pltpu.sync_copy(za_hbm, zbuf_v)
    for k in range(rp // _RZ):
      pltpu.sync_copy(zbuf_v, acc_sh.at[pl.ds(s * rp + k * _RZ, _RZ)])
    if with_count:
      pltpu.sync_copy(zc_hbm, zcnt_v)
      pltpu.sync_copy(zcnt_v, cnt_sh.at[pl.ds(s * rp, rp)])
      pltpu.sync_copy(oc_hbm, ones_v)
    plsc.subcore_barrier()

    def step(jj, carry):
      pltpu.async_copy(x_hbm.at[src_v.at[jj]], rows_v, sem).wait()
      pltpu.sync_copy(rows_v, acc_sh.at[dst_v.at[jj]], add=True)
      if with_count:
        pltpu.sync_copy(ones_v, cnt_sh.at[dst_v.at[jj]], add=True)
      return carry
    lax.fori_loop(0, j_chunks, step, 0)

    plsc.subcore_barrier()
    for k in range(rp // _RZ):
      pltpu.sync_copy(acc_sh.at[pl.ds(s * rp + k * _RZ, _RZ)], zbuf_v)
      pltpu.sync_copy(zbuf_v, sums_hbm.at[c, pl.ds(s * rp + k * _RZ, _RZ)])
    if with_count:
      pltpu.sync_copy(cnt_sh.at[pl.ds(s * rp, rp)], zcnt_v)
      pltpu.sync_copy(zcnt_v, cnts_hbm.at[c, pl.ds(s * rp, rp)])

  return pl.kernel(body, out_type=out_type, mesh=mesh, scratch_types=scratch)


def _tc_body(sums_ref, cnts_ref, x_ref, wl_ref, wr_ref, bl_ref, br_ref, o_ref):
  ssum = sums_ref[0] + sums_ref[1]
  cnt = cnts_ref[0] + cnts_ref[1]
  mean = ssum / jnp.maximum(cnt, 1.0)
  acc = lax.dot_general(mean, wl_ref[...], (((1,), (1,)), ((), ())),
                        preferred_element_type=jnp.float32)
  acc = acc + lax.dot_general(x_ref[...], wr_ref[...], (((1,), (1,)), ((), ())),
                              preferred_element_type=jnp.float32)
  o_ref[...] = jnp.maximum(acc + bl_ref[...] + br_ref[...], 0.0)


def _combine(sums, cnts, x, wl, wr, bl, br):
  n, f = x.shape
  br_rows = 1000
  grid = (n // br_rows,)
  return pl.pallas_call(
      _tc_body,
      grid=grid,
      in_specs=[
          pl.BlockSpec((_NC, br_rows, f), lambda i: (0, i, 0)),
          pl.BlockSpec((_NC, br_rows, 1), lambda i: (0, i, 0)),
          pl.BlockSpec((br_rows, f), lambda i: (i, 0)),
          pl.BlockSpec((f, f), lambda i: (0, 0)),
          pl.BlockSpec((f, f), lambda i: (0, 0)),
          pl.BlockSpec((1, f), lambda i: (0, 0)),
          pl.BlockSpec((1, f), lambda i: (0, 0)),
      ],
      out_specs=pl.BlockSpec((br_rows, f), lambda i: (i, 0)),
      out_shape=jax.ShapeDtypeStruct((n, f), jnp.float32),
  )(sums, cnts, x, wl, wr, bl, br)


def kernel(x, edge_index, Wl0, bl0, Wr0, br0, Wl1, bl1, Wr1, br1):
  n, f = x.shape
  e = edge_index.shape[1]
  nw = _NC * _NS
  j_chunks = -(-e // (nw * _C))
  ep = nw * _C * j_chunks
  n_pad = -(-n // (_NS * _RZ)) * (_NS * _RZ)

  pad = ep - e
  src = jnp.concatenate([edge_index[0], jnp.zeros((pad,), jnp.int32)])
  dst = jnp.concatenate([edge_index[1], jnp.full((pad,), n, jnp.int32)])
  srcs = src.reshape(nw, j_chunks, _C)
  dsts = dst.reshape(nw, j_chunks, _C)

  za = jnp.zeros((_RZ, f), jnp.float32)
  zc = jnp.zeros((n_pad // _NS,), jnp.float32)
  oc = jnp.ones((_C,), jnp.float32)

  agg_c = _make_agg(n_pad, f, j_chunks, True)
  sums, cnts = agg_c(x, srcs, dsts, za, zc, oc)
  cnts = cnts.reshape(_NC, n_pad, 1)
  h = _combine(sums, cnts, x, Wl0, Wr0, bl0.reshape(1, f), br0.reshape(1, f))

  agg = _make_agg(n_pad, f, j_chunks, False)
  (sums2,) = agg(h, srcs, dsts, za)
  out = _combine(sums2, cnts, h, Wl1, Wr1, bl1.reshape(1, f), br1.reshape(1, f))
  return out
